# Initial kernel scaffold; baseline (speedup 1.0000x reference)
import jax
import jax.numpy as jnp
from jax.experimental import pallas as pl


def kernel(ip_features, port_indices, proto_indices, W_ip, b_ip, port_table, proto_table):
    raise NotImplementedError("write your pallas kernel here")


# speedup vs baseline: 1.0808x; 1.0808x over previous
import functools

import jax
import jax.numpy as jnp
from jax import lax
from jax.experimental import pallas as pl
from jax.experimental.pallas import tpu as pltpu
from jax.experimental.pallas import tpu_sc as plsc

N = 16384
D = 64
_INFO = plsc.get_sparse_core_info()
_NC, _NS = _INFO.num_cores, _INFO.num_subcores
_NW = _NC * _NS
_BPW = N // _NW

_MESH = plsc.VectorSubcoreMesh(core_axis_name="c", subcore_axis_name="s")


@functools.partial(
    pl.kernel,
    mesh=_MESH,
    compiler_params=pltpu.CompilerParams(use_tc_tiling_on_sc=False),
    out_type=[
        jax.ShapeDtypeStruct((N, D), jnp.float32),
        jax.ShapeDtypeStruct((N, D), jnp.float32),
    ],
    scratch_types=[
        pltpu.VMEM((_BPW,), jnp.int32),
        pltpu.VMEM((_BPW, D), jnp.float32),
        pltpu.VMEM((_BPW,), jnp.int32),
        pltpu.VMEM((_BPW, D), jnp.float32),
        pltpu.SemaphoreType.DMA,
        pltpu.SemaphoreType.DMA,
    ],
)
def _sc_gather(port_table, port_idx, proto_table, proto_idx,
               out_port, out_proto,
               pidx_v, prow_v, qidx_v, qrow_v, psem, qsem):
    wid = lax.axis_index("s") * _NC + lax.axis_index("c")
    base = wid * _BPW
    pltpu.sync_copy(port_idx.at[pl.ds(base, _BPW)], pidx_v)
    pltpu.sync_copy(proto_idx.at[pl.ds(base, _BPW)], qidx_v)
    pcopy = pltpu.async_copy(port_table.at[pidx_v], prow_v, psem)
    qcopy = pltpu.async_copy(proto_table.at[qidx_v], qrow_v, qsem)
    pcopy.wait()
    qcopy.wait()
    pltpu.sync_copy(prow_v, out_port.at[pl.ds(base, _BPW)])
    pltpu.sync_copy(qrow_v, out_proto.at[pl.ds(base, _BPW)])


def _ip_body(x_ref, w_ref, b_ref, o_ref):
    o_ref[...] = (
        jnp.dot(x_ref[...], w_ref[...], preferred_element_type=jnp.float32)
        + b_ref[...]
    )


_IP_BLK = 2048


def _ip_linear(ip_features, W_ip, b_ip):
    return pl.pallas_call(
        _ip_body,
        grid=(N // _IP_BLK,),
        in_specs=[
            pl.BlockSpec((_IP_BLK, 32), lambda i: (i, 0)),
            pl.BlockSpec((32, D), lambda i: (0, 0)),
            pl.BlockSpec((1, D), lambda i: (0, 0)),
        ],
        out_specs=pl.BlockSpec((_IP_BLK, D), lambda i: (i, 0)),
        out_shape=jax.ShapeDtypeStruct((N, D), jnp.float32),
    )(ip_features, W_ip, b_ip.reshape(1, D))


def kernel(ip_features, port_indices, proto_indices, W_ip, b_ip,
           port_table, proto_table):
    x_ip = _ip_linear(ip_features, W_ip, b_ip)
    x_port, x_proto = _sc_gather(
        port_table, port_indices.astype(jnp.int32),
        proto_table, proto_indices.astype(jnp.int32))
    return (x_ip, x_port, x_proto)
